# baseline (speedup 1.0000x reference)
import math

import jax
import jax.numpy as jnp
from jax.experimental import pallas as pl
from jax.experimental.pallas import tpu as pltpu

_B, _H, _S, _D = 1, 16, 2048, 128
_BM = 192
_TOPK = 512
_RAND_P = 0.01
_LOCAL_W = 128
_G = -(-_S // _BM)
_SP = _G * _BM
_CH = 256


def _fused_kernel(q_ref, k_ref, v_ref, rm_ref, o_ref, oc_ref):
    g = pl.program_id(1)
    q = q_ref[0, 0]
    k = k_ref[0, 0]
    v = v_ref[0, 0]
    scale = 1.0 / math.sqrt(_D)

    logits = jax.lax.dot_general(
        q.astype(jnp.bfloat16), k.astype(jnp.bfloat16),
        (((1,), (1,)), ((), ())),
        preferred_element_type=jnp.float32) * scale
    m = jnp.max(logits, axis=-1, keepdims=True)
    p = jnp.exp(logits - m)
    l = jnp.sum(p, axis=-1, keepdims=True)
    vb = v.astype(jnp.bfloat16)

    rowT = jax.lax.broadcasted_iota(jnp.int32, (1, _BM), 1)
    validT = ((g * _BM + rowT) < _S).astype(jnp.bfloat16)
    probs_bf = (p / l).astype(jnp.bfloat16)
    bs = jax.lax.dot_general(
        validT, probs_bf, (((1,), (0,)), ((), ())),
        preferred_element_type=jnp.float32)

    kidx = jax.lax.broadcasted_iota(jnp.int32, (1, _S), 1)
    bs_i = jax.lax.bitcast_convert_type(bs, jnp.int32)
    k16 = jax.lax.broadcasted_iota(jnp.int32, (16, 1), 0) + 1
    lo = jnp.zeros((1, 1), jnp.int32)
    hi = jnp.max(bs_i, keepdims=True) + 1
    for _ in range(9):
        step = (hi - lo + 15) >> 4
        thr = lo + k16 * step
        cnt = jnp.sum((bs_i >= thr).astype(jnp.int32), axis=1, keepdims=True)
        s = jnp.sum((cnt >= _TOPK).astype(jnp.int32), keepdims=True)
        lo = lo + s * step
        hi = lo + step
    t = lo
    c_gt = jnp.sum((bs_i >= t + 1).astype(jnp.int32), keepdims=True)
    quota = _TOPK - c_gt
    eq = bs_i == t
    ilo = jnp.full((1, 1), -1, jnp.int32)
    ihi = jnp.full((1, 1), _S - 1, jnp.int32)
    for _ in range(3):
        step = (ihi - ilo + 15) >> 4
        thr = ilo + k16 * step
        cnt = jnp.sum((eq & (kidx <= thr)).astype(jnp.int32), axis=1, keepdims=True)
        s = jnp.sum((cnt < quota).astype(jnp.int32), keepdims=True)
        ilo = ilo + s * step
        ihi = ilo + step
    topk_mask = (bs_i >= t + 1) | (eq & (kidx <= ihi))

    o = jax.lax.dot_general(
        p.astype(jnp.bfloat16), vb, (((1,), (0,)), ((), ())),
        preferred_element_type=jnp.float32) / l
    o_ref[0, 0] = o

    gstart = g * _BM
    static = (kidx >= gstart - _LOCAL_W) & (kidx < gstart + _BM + _LOCAL_W)
    rmask = rm_ref[0, 0] != 0
    mask = topk_mask | rmask | static

    w = jnp.where(mask, p, 0.0)
    pmax = jnp.max(w, axis=-1, keepdims=True)
    p2 = w * (1.0 / pmax)
    l2 = jnp.sum(p2, axis=-1, keepdims=True)
    o_sparse = jax.lax.dot_general(
        p2.astype(jnp.bfloat16), vb, (((1,), (0,)), ((), ())),
        preferred_element_type=jnp.float32) / l2
    oc_ref[0, 0] = o - o_sparse


def kernel(q, k, v):
    b, h, s, d = q.shape
    rkey = jax.random.fold_in(jax.random.key(1), 7)
    rmask = (jax.random.uniform(rkey, (b, h, _G, s)) < _RAND_P)
    rmask = rmask[0].reshape(h, _G, 1, s).astype(jnp.int32)

    qp = jnp.pad(q, ((0, 0), (0, 0), (0, _SP - s), (0, 0)))

    o, oc = pl.pallas_call(
        _fused_kernel,
        grid=(h, _G),
        in_specs=[
            pl.BlockSpec((1, 1, _BM, d), lambda hh, gg: (0, hh, gg, 0)),
            pl.BlockSpec((1, 1, s, d), lambda hh, gg: (0, hh, 0, 0)),
            pl.BlockSpec((1, 1, s, d), lambda hh, gg: (0, hh, 0, 0)),
            pl.BlockSpec((1, 1, 1, s), lambda hh, gg: (hh, gg, 0, 0)),
        ],
        out_specs=[
            pl.BlockSpec((1, 1, _BM, d), lambda hh, gg: (0, hh, gg, 0)),
            pl.BlockSpec((1, 1, _BM, d), lambda hh, gg: (0, hh, gg, 0)),
        ],
        out_shape=[
            jax.ShapeDtypeStruct((b, h, _SP, d), jnp.float32),
            jax.ShapeDtypeStruct((b, h, _SP, d), jnp.float32),
        ],
        compiler_params=pltpu.CompilerParams(
            dimension_semantics=("parallel", "parallel")),
    )(qp, k, v, rmask)

    return jnp.stack([o[:, :, :s], oc[:, :, :s]], axis=0)


# speedup vs baseline: 1.8783x; 1.8783x over previous
import math

import jax
import jax.numpy as jnp
from jax.experimental import pallas as pl
from jax.experimental.pallas import tpu as pltpu

_B, _H, _S, _D = 1, 16, 2048, 128
_BM = 192
_TOPK = 512
_RAND_P = 0.01
_LOCAL_W = 128
_G = -(-_S // _BM)
_GPP = 4
_NJ = 3
_G2 = _GPP * _NJ
_BMG = _GPP * _BM
_SP = _G2 * _BM


def _fused_kernel(q_ref, k_ref, v_ref, rm_ref, o_ref, oc_ref):
    j = pl.program_id(1)
    q = q_ref[0, 0]
    k = k_ref[0, 0]
    v = v_ref[0, 0]
    scale = 1.0 / math.sqrt(_D)

    logits = jax.lax.dot_general(
        q.astype(jnp.bfloat16), k.astype(jnp.bfloat16),
        (((1,), (1,)), ((), ())),
        preferred_element_type=jnp.float32) * scale
    m = jnp.max(logits, axis=-1, keepdims=True)
    p = jnp.exp(logits - m)
    l = jnp.sum(p, axis=-1, keepdims=True)
    vb = v.astype(jnp.bfloat16)

    col = jax.lax.broadcasted_iota(jnp.int32, (_GPP, _BMG), 1)
    grp = jax.lax.broadcasted_iota(jnp.int32, (_GPP, _BMG), 0)
    valid4 = (((col // _BM) == grp) &
              ((j * _BMG + col) < _S)).astype(jnp.bfloat16)
    probs_bf = (p / l).astype(jnp.bfloat16)
    bs = jax.lax.dot_general(
        valid4, probs_bf, (((1,), (0,)), ((), ())),
        preferred_element_type=jnp.float32)

    bs_i = jax.lax.bitcast_convert_type(bs, jnp.int32).reshape(_GPP, 1, _S)
    kidx = jax.lax.broadcasted_iota(jnp.int32, (1, 1, _S), 2)
    k16 = jax.lax.broadcasted_iota(jnp.int32, (1, 16, 1), 1) + 1
    lo = jnp.zeros((_GPP, 1, 1), jnp.int32)
    hi = jnp.max(bs_i, axis=2, keepdims=True) + 1
    for _ in range(9):
        step = (hi - lo + 15) >> 4
        thr = lo + k16 * step
        cnt = jnp.sum((bs_i >= thr).astype(jnp.int32), axis=2, keepdims=True)
        s = jnp.sum((cnt >= _TOPK).astype(jnp.int32), axis=1, keepdims=True)
        lo = lo + s * step
        hi = lo + step
    t = lo
    c_gt = jnp.sum((bs_i >= t + 1).astype(jnp.int32), axis=2, keepdims=True)
    quota = _TOPK - c_gt
    eq = bs_i == t
    ilo = jnp.full((_GPP, 1, 1), -1, jnp.int32)
    ihi = jnp.full((_GPP, 1, 1), _S - 1, jnp.int32)
    for _ in range(3):
        step = (ihi - ilo + 15) >> 4
        thr = ilo + k16 * step
        cnt = jnp.sum((eq & (kidx <= thr)).astype(jnp.int32), axis=2, keepdims=True)
        s = jnp.sum((cnt < quota).astype(jnp.int32), axis=1, keepdims=True)
        ilo = ilo + s * step
        ihi = ilo + step
    topk_mask = (bs_i >= t + 1) | (eq & (kidx <= ihi))

    gstart = (j * _GPP + jax.lax.broadcasted_iota(
        jnp.int32, (_GPP, 1, 1), 0)) * _BM
    static = (kidx >= gstart - _LOCAL_W) & (kidx < gstart + _BM + _LOCAL_W)
    rmask = (rm_ref[0, 0] != 0).reshape(_GPP, 1, _S)
    mask = topk_mask | rmask | static

    p3 = p.reshape(_GPP, _BM, _S)
    w3 = jnp.where(mask, p3, 0.0)
    pmax = jnp.max(w3, axis=2, keepdims=True)
    p23 = w3 * (1.0 / pmax)
    l2 = jnp.sum(p23, axis=2, keepdims=True).reshape(_BMG, 1)
    o_sparse = jax.lax.dot_general(
        p23.reshape(_BMG, _S).astype(jnp.bfloat16), vb,
        (((1,), (0,)), ((), ())),
        preferred_element_type=jnp.float32) / l2
    o = jax.lax.dot_general(
        p.astype(jnp.bfloat16), vb, (((1,), (0,)), ((), ())),
        preferred_element_type=jnp.float32) / l
    o_ref[0, 0] = o
    oc_ref[0, 0] = o - o_sparse


def kernel(q, k, v):
    b, h, s, d = q.shape
    rkey = jax.random.fold_in(jax.random.key(1), 7)
    rmask = (jax.random.uniform(rkey, (b, h, _G, s)) < _RAND_P)
    rmask = jnp.pad(rmask[0].astype(jnp.int32), ((0, 0), (0, _G2 - _G), (0, 0)))
    rmask = rmask.reshape(h, _NJ, _GPP, s)

    qp = jnp.pad(q, ((0, 0), (0, 0), (0, _SP - s), (0, 0)))

    o, oc = pl.pallas_call(
        _fused_kernel,
        grid=(h, _NJ),
        in_specs=[
            pl.BlockSpec((1, 1, _BMG, d), lambda hh, jj: (0, hh, jj, 0)),
            pl.BlockSpec((1, 1, s, d), lambda hh, jj: (0, hh, 0, 0)),
            pl.BlockSpec((1, 1, s, d), lambda hh, jj: (0, hh, 0, 0)),
            pl.BlockSpec((1, 1, _GPP, s), lambda hh, jj: (hh, jj, 0, 0)),
        ],
        out_specs=[
            pl.BlockSpec((1, 1, _BMG, d), lambda hh, jj: (0, hh, jj, 0)),
            pl.BlockSpec((1, 1, _BMG, d), lambda hh, jj: (0, hh, jj, 0)),
        ],
        out_shape=[
            jax.ShapeDtypeStruct((b, h, _SP, d), jnp.float32),
            jax.ShapeDtypeStruct((b, h, _SP, d), jnp.float32),
        ],
        compiler_params=pltpu.CompilerParams(
            dimension_semantics=("parallel", "parallel")),
    )(qp, k, v, rmask)

    return jnp.stack([o[:, :, :s], oc[:, :, :s]], axis=0)


# speedup vs baseline: 2.1463x; 1.1427x over previous
import math

import jax
import jax.numpy as jnp
from jax.experimental import pallas as pl
from jax.experimental.pallas import tpu as pltpu

_B, _H, _S, _D = 1, 16, 2048, 128
_BM = 192
_TOPK = 512
_RAND_P = 0.01
_LOCAL_W = 128
_G = -(-_S // _BM)
_GPP = 6
_NJ = 2
_G2 = _GPP * _NJ
_BMG = _GPP * _BM
_SP = _G2 * _BM


def _fused_kernel(q_ref, k_ref, v_ref, rm_ref, vg_ref, o_ref, oc_ref):
    j = pl.program_id(1)
    q = q_ref[0, 0]
    k = k_ref[0, 0]
    v = v_ref[0, 0]
    scale = 1.0 / math.sqrt(_D)

    logits = jax.lax.dot_general(
        q.astype(jnp.bfloat16), k.astype(jnp.bfloat16),
        (((1,), (1,)), ((), ())),
        preferred_element_type=jnp.float32) * scale
    m = jnp.max(logits, axis=-1, keepdims=True)
    p = jnp.exp(logits - m)
    l = jnp.sum(p, axis=-1, keepdims=True)
    vb = v.astype(jnp.bfloat16)

    probs_bf = (p / l).astype(jnp.bfloat16)
    bs = jax.lax.dot_general(
        vg_ref[0], probs_bf, (((1,), (0,)), ((), ())),
        preferred_element_type=jnp.float32)

    bs_i = jax.lax.bitcast_convert_type(bs, jnp.int32).reshape(_GPP, 1, _S)
    kidx = jax.lax.broadcasted_iota(jnp.int32, (1, 1, _S), 2)
    k16 = jax.lax.broadcasted_iota(jnp.int32, (1, 16, 1), 1) + 1
    lo = jnp.zeros((_GPP, 1, 1), jnp.int32)
    hi = jnp.max(bs_i, axis=2, keepdims=True) + 1
    for _ in range(9):
        step = (hi - lo + 15) >> 4
        thr = lo + k16 * step
        cnt = jnp.sum((bs_i >= thr).astype(jnp.int32), axis=2, keepdims=True)
        s = jnp.sum((cnt >= _TOPK).astype(jnp.int32), axis=1, keepdims=True)
        lo = lo + s * step
        hi = lo + step
    t = lo
    c_gt = jnp.sum((bs_i >= t + 1).astype(jnp.int32), axis=2, keepdims=True)
    quota = _TOPK - c_gt
    eq = bs_i == t
    ilo = jnp.full((_GPP, 1, 1), -1, jnp.int32)
    ihi = jnp.full((_GPP, 1, 1), _S - 1, jnp.int32)
    for _ in range(3):
        step = (ihi - ilo + 15) >> 4
        thr = ilo + k16 * step
        cnt = jnp.sum((eq & (kidx <= thr)).astype(jnp.int32), axis=2, keepdims=True)
        s = jnp.sum((cnt < quota).astype(jnp.int32), axis=1, keepdims=True)
        ilo = ilo + s * step
        ihi = ilo + step
    topk_mask = (bs_i >= t + 1) | (eq & (kidx <= ihi))

    gstart = (j * _GPP + jax.lax.broadcasted_iota(
        jnp.int32, (_GPP, 1, 1), 0)) * _BM
    static = (kidx >= gstart - _LOCAL_W) & (kidx < gstart + _BM + _LOCAL_W)
    rmask = (rm_ref[0, 0] != 0).reshape(_GPP, 1, _S)
    mask = topk_mask | rmask | static

    p3 = p.reshape(_GPP, _BM, _S)
    w3 = jnp.where(mask, p3, 0.0)
    l2 = jnp.sum(w3, axis=2, keepdims=True).reshape(_BMG, 1)
    o_sparse = jax.lax.dot_general(
        w3.reshape(_BMG, _S).astype(jnp.bfloat16), vb,
        (((1,), (0,)), ((), ())),
        preferred_element_type=jnp.float32) / l2
    o = jax.lax.dot_general(
        p.astype(jnp.bfloat16), vb, (((1,), (0,)), ((), ())),
        preferred_element_type=jnp.float32) / l
    o_ref[0, 0] = o
    oc_ref[0, 0] = o - o_sparse


def kernel(q, k, v):
    b, h, s, d = q.shape
    rkey = jax.random.fold_in(jax.random.key(1), 7)
    rmask = (jax.random.uniform(rkey, (b, h, _G, s)) < _RAND_P)
    rmask = jnp.pad(rmask[0].astype(jnp.int32), ((0, 0), (0, _G2 - _G), (0, 0)))
    rmask = rmask.reshape(h, _NJ, _GPP, s)

    col = jnp.arange(_BMG)[None, None, :]
    grp = jnp.arange(_GPP)[None, :, None]
    jj = jnp.arange(_NJ)[:, None, None]
    vgrp = (((col // _BM) == grp) &
            ((jj * _BMG + col) < s)).astype(jnp.bfloat16)

    qp = jnp.pad(q, ((0, 0), (0, 0), (0, _SP - s), (0, 0)))

    o, oc = pl.pallas_call(
        _fused_kernel,
        grid=(h, _NJ),
        in_specs=[
            pl.BlockSpec((1, 1, _BMG, d), lambda hh, jj: (0, hh, jj, 0)),
            pl.BlockSpec((1, 1, s, d), lambda hh, jj: (0, hh, 0, 0)),
            pl.BlockSpec((1, 1, s, d), lambda hh, jj: (0, hh, 0, 0)),
            pl.BlockSpec((1, 1, _GPP, s), lambda hh, jj: (hh, jj, 0, 0)),
            pl.BlockSpec((1, _GPP, _BMG), lambda hh, jj: (jj, 0, 0)),
        ],
        out_specs=[
            pl.BlockSpec((1, 1, _BMG, d), lambda hh, jj: (0, hh, jj, 0)),
            pl.BlockSpec((1, 1, _BMG, d), lambda hh, jj: (0, hh, jj, 0)),
        ],
        out_shape=[
            jax.ShapeDtypeStruct((b, h, _SP, d), jnp.float32),
            jax.ShapeDtypeStruct((b, h, _SP, d), jnp.float32),
        ],
        compiler_params=pltpu.CompilerParams(
            dimension_semantics=("parallel", "parallel")),
    )(qp, k, v, rmask, vgrp)

    return jnp.stack([o[:, :, :s], oc[:, :, :s]], axis=0)
